# baseline (speedup 1.0000x reference)
import functools

import jax
import jax.numpy as jnp
from jax import lax
from jax.experimental import pallas as pl
from jax.experimental.pallas import tpu as pltpu
from jax.experimental.pallas import tpu_sc as plsc

_LANES = 16


@functools.lru_cache(maxsize=None)
def _build(B, C, V, D):
    info = plsc.get_sparse_core_info()
    NC, NS = info.num_cores, info.num_subcores
    NW = NC * NS
    R = B // NW
    NV = R // _LANES
    assert B % NW == 0 and R % _LANES == 0

    mesh = plsc.VectorSubcoreMesh(core_axis_name="c", subcore_axis_name="s")

    @functools.partial(
        pl.kernel,
        mesh=mesh,
        out_type=jax.ShapeDtypeStruct((B, C * D), jnp.float32),
        compiler_params=pltpu.CompilerParams(use_tc_tiling_on_sc=False),
        scratch_types=[
            pltpu.VMEM((C * R,), jnp.int32),
            [pltpu.VMEM((R,), jnp.int32) for _ in range(2)],
            [pltpu.VMEM((R, D), jnp.float32) for _ in range(2)],
            pltpu.SemaphoreType.DMA,
            [pltpu.SemaphoreType.DMA for _ in range(2)],
            [pltpu.SemaphoreType.DMA for _ in range(2)],
        ],
    )
    def gather_kernel(xcm_hbm, tab_hbm, out_hbm, xtv, fvs, rowss, semi,
                      semg, semo):
        wid = lax.axis_index("s") * NC + lax.axis_index("c")
        base = wid * R
        idx_cps = [
            pltpu.async_copy(
                xcm_hbm.at[c].at[pl.ds(pl.multiple_of(base, 8), R)],
                xtv.at[pl.ds(c * R, R)],
                semi,
            )
            for c in range(C)
        ]
        for cp in idx_cps:
            cp.wait()
        gather_cps = [None] * C
        out_cps = [None] * C

        def compute_fv(c):
            fv = fvs[c % 2]
            off = c * V
            for t in range(NV):
                s = pl.ds(t * _LANES, _LANES)
                fv[s] = xtv[pl.ds(c * R + t * _LANES, _LANES)] + off
            return fv

        def store_out(c):
            return pltpu.async_copy(
                rowss[c % 2],
                out_hbm.at[pl.ds(base, R), pl.ds(c * D, D)],
                semo[c % 2],
            )

        for c in range(C):
            if c >= 2:
                out_cps[c - 2].wait()
            fv = compute_fv(c)
            gather_cps[c] = pltpu.async_copy(
                tab_hbm.at[fv], rowss[c % 2], semg[c % 2])
            if c > 0:
                gather_cps[c - 1].wait()
                out_cps[c - 1] = store_out(c - 1)
        gather_cps[C - 1].wait()
        out_cps[C - 2].wait()
        out_cps[C - 1] = store_out(C - 1)
        out_cps[C - 1].wait()

    return gather_kernel


def kernel(x_batch, tables):
    B, C = x_batch.shape
    _, V, D = tables.shape
    xt = x_batch.T
    tab = tables.reshape(C * V, D)
    return _build(B, C, V, D)(xt, tab)


# speedup vs baseline: 1.0089x; 1.0089x over previous
import functools

import jax
import jax.numpy as jnp
from jax import lax
from jax.experimental import pallas as pl
from jax.experimental.pallas import tpu as pltpu
from jax.experimental.pallas import tpu_sc as plsc

_LANES = 16


@functools.lru_cache(maxsize=None)
def _build(B, C, V, D):
    info = plsc.get_sparse_core_info()
    NC, NS = info.num_cores, info.num_subcores
    NW = NC * NS
    R = B // NW
    CR = 128
    NCH = R // CR
    NIDX = CR * C
    NVEC = NIDX // _LANES
    assert B % NW == 0 and R % CR == 0

    mesh = plsc.VectorSubcoreMesh(core_axis_name="c", subcore_axis_name="s")

    @functools.partial(
        pl.kernel,
        mesh=mesh,
        out_type=jax.ShapeDtypeStruct((B * C, D), jnp.float32),
        compiler_params=pltpu.CompilerParams(use_tc_tiling_on_sc=False),
        scratch_types=[
            pltpu.VMEM((NIDX,), jnp.int32),
            pltpu.VMEM((NIDX,), jnp.int32),
            pltpu.VMEM((NIDX, D), jnp.float32),
            pltpu.SemaphoreType.DMA,
        ],
    )
    def gather_kernel(x_hbm, tab_hbm, out_hbm, xv, fv, rows, sem):
        wid = lax.axis_index("s") * NC + lax.axis_index("c")
        lane = lax.iota(jnp.int32, _LANES)

        def chunk(j, carry):
            p0 = pl.multiple_of((wid * NCH + j) * NIDX, 8)
            pltpu.sync_copy(x_hbm.at[pl.ds(p0, NIDX)], xv)
            for t in range(NVEC):
                col = (lane + (t * _LANES)) % C
                fv[pl.ds(t * _LANES, _LANES)] = (
                    xv[pl.ds(t * _LANES, _LANES)] + col * V
                )
            pltpu.async_copy(tab_hbm.at[fv], rows, sem).wait()
            pltpu.sync_copy(rows, out_hbm.at[pl.ds(p0, NIDX)])
            return carry

        lax.fori_loop(0, NCH, chunk, 0)

    return gather_kernel


def kernel(x_batch, tables):
    B, C = x_batch.shape
    _, V, D = tables.shape
    x_flat = x_batch.reshape(B * C)
    tab = tables.reshape(C * V, D)
    out = _build(B, C, V, D)(x_flat, tab)
    return out.reshape(B, C * D)
